# baseline (speedup 1.0000x reference)
import functools
import jax
import jax.numpy as jnp
from jax import lax
from jax.experimental import pallas as pl
from jax.experimental.pallas import tpu as pltpu
from jax.experimental.pallas import tpu_sc as plsc

NC = 2
NS = 16
NW = NC * NS

N_PAD = 10240
ZR = N_PAD // NS
BLK = 1024
GW = 16


def _sc_mesh():
    return plsc.VectorSubcoreMesh(
        core_axis_name="c", subcore_axis_name="s", num_cores=NC, num_subcores=NS
    )


def _aggregate(gs, src, dst, e_chunk, n_chunks, mode="feature"):
    nq = len(gs)
    w = gs[0].shape[-1]
    per_core = nq // NC if mode == "feature" else 1
    n_out = nq if mode == "feature" else NC
    e_src = src.shape[0]
    e_half = e_src // NC
    zeros = jnp.zeros((ZR, w), jnp.float32)

    @functools.partial(
        pl.kernel,
        out_type=jax.ShapeDtypeStruct((n_out, N_PAD, w), jnp.float32),
        mesh=_sc_mesh(),
        scratch_types=[
            pltpu.VMEM((2, e_chunk), jnp.int32),
            pltpu.VMEM((2, e_chunk), jnp.int32),
            pltpu.VMEM((2, e_chunk, w), jnp.float32),
            pltpu.VMEM_SHARED((N_PAD, w), jnp.float32),
            pltpu.SemaphoreType.DMA,
            pltpu.SemaphoreType.DMA,
        ],
        compiler_params=pltpu.CompilerParams(use_tc_tiling_on_sc=False),
    )
    def agg_kernel(*refs):
        g_hbms = refs[:nq]
        (src_hbm, dst_hbm, z_hbm, out_hbm, idx_s, idx_d, rows_v, acc, sem0,
         sem1) = refs[nq:]
        sems = (sem0, sem1)
        c = lax.axis_index("c")
        s = lax.axis_index("s")
        base = s * (e_chunk * n_chunks)

        def offs(cc, ch):
            off = base + ch * e_chunk
            if mode == "edge":
                return cc * e_half + off, cc * e_half + off
            if mode == "role":
                return off, off
            return off, off

        for t in range(per_core):
            pltpu.sync_copy(z_hbm, acc.at[pl.ds(s * ZR, ZR)])
            plsc.subcore_barrier()
            for cc in range(NC):
                @pl.when(c == cc)
                def _():
                    s_off, d_off = offs(cc, 0)
                    pltpu.sync_copy(src_hbm.at[pl.ds(s_off, e_chunk)],
                                    idx_s.at[0])
                    d_ref0 = src_hbm if (mode == "role" and cc == 1) \
                        else dst_hbm
                    pltpu.sync_copy(d_ref0.at[pl.ds(d_off, e_chunk)],
                                    idx_d.at[0])
                    g = g_hbms[(cc * per_core + t) % nq]
                    pltpu.async_copy(g.at[idx_s.at[0]], rows_v.at[0],
                                     sems[0])
            for ch in range(n_chunks):
                b = 0 if mode == "role" else ch % 2
                nb = 1 - (ch % 2)
                for cc in range(NC):
                    @pl.when(c == cc)
                    def _():
                        g = g_hbms[(cc * per_core + t) % nq]
                        if mode == "role":
                            if ch == 0:
                                pltpu.make_async_copy(
                                    g.at[idx_s.at[0]], rows_v.at[0],
                                    sems[0]).wait()
                            if ch + 1 < n_chunks:
                                d_ref = dst_hbm if cc == 0 else src_hbm
                                off1 = base + (ch + 1) * e_chunk
                                pltpu.sync_copy(
                                    d_ref.at[pl.ds(off1, e_chunk)],
                                    idx_d.at[nb])
                        else:
                            if ch + 1 < n_chunks:
                                s_off, d_off = offs(cc, ch + 1)
                                pltpu.sync_copy(
                                    src_hbm.at[pl.ds(s_off, e_chunk)],
                                    idx_s.at[nb])
                                pltpu.sync_copy(
                                    dst_hbm.at[pl.ds(d_off, e_chunk)],
                                    idx_d.at[nb])
                                pltpu.async_copy(g.at[idx_s.at[nb]],
                                                 rows_v.at[nb], sems[nb])
                            pltpu.make_async_copy(g.at[idx_s.at[b]],
                                                  rows_v.at[b],
                                                  sems[b]).wait()

                db = b if mode != "role" else ch % 2
                pltpu.sync_copy(rows_v.at[b], acc.at[idx_d.at[db]],
                                add=True)
            plsc.subcore_barrier()
            for cc in range(NC):
                @pl.when(c == cc)
                def _():
                    oq = cc * per_core + t if mode == "feature" else cc
                    pltpu.sync_copy(acc.at[pl.ds(s * ZR, ZR)],
                                    out_hbm.at[oq, pl.ds(s * ZR, ZR)])

            if t + 1 < per_core:
                plsc.subcore_barrier()

    return agg_kernel(*gs, src, dst, zeros)


def _norms_from_dp(dp_ref):
    nd = lax.rsqrt(dp_ref[0, :, 0:1] + 1.0)
    ns = lax.rsqrt(dp_ref[1, :, 0:1] + 1.0)
    return nd, ns


def _grouped(o_refs, g):
    w = g.shape[-1] // len(o_refs)
    for q, o in enumerate(o_refs):
        o[...] = g[:, q * w:(q + 1) * w]


def _tc_first(xp, dp, w1):
    in_dim, d_out = w1.shape
    nq = d_out // 32

    def body(x_ref, dp_ref, w_ref, *o_refs):
        _, ns = _norms_from_dp(dp_ref)
        g = jnp.dot(x_ref[...] * ns, w_ref[...],
                    preferred_element_type=jnp.float32)
        _grouped(o_refs, g)

    return pl.pallas_call(
        body,
        grid=(N_PAD // BLK,),
        in_specs=[
            pl.BlockSpec((BLK, in_dim), lambda i: (i, 0)),
            pl.BlockSpec((2, BLK, GW), lambda i: (0, i, 0)),
            pl.BlockSpec((in_dim, d_out), lambda i: (0, 0)),
        ],
        out_specs=[pl.BlockSpec((BLK, 32), lambda i: (i, 0))] * nq,
        out_shape=[jax.ShapeDtypeStruct((N_PAD, 32), jnp.float32)] * nq,
    )(xp, dp, w1)


def _tc_mid(p, gs, dp, w, b8):
    d_in, d_out = w.shape
    nq_in = len(gs)
    wi = d_in // nq_in
    nq_out = d_out // 32

    def body(p_ref, *refs):
        g_refs = refs[:nq_in]
        dp_ref, w_ref, b_ref = refs[nq_in:nq_in + 3]
        o_refs = refs[nq_in + 3:]
        nd, ns = _norms_from_dp(dp_ref)
        agg = jnp.concatenate(
            [p_ref[q] + g_refs[q][...] for q in range(nq_in)], axis=1)
        h = jnp.maximum(agg * nd + b_ref[0:1, :], 0.0)
        g = jnp.dot(h * ns, w_ref[...], preferred_element_type=jnp.float32)
        _grouped(o_refs, g)

    return pl.pallas_call(
        body,
        grid=(N_PAD // BLK,),
        in_specs=[pl.BlockSpec((nq_in, BLK, wi), lambda i: (0, i, 0))]
        + [pl.BlockSpec((BLK, wi), lambda i: (i, 0))] * nq_in
        + [
            pl.BlockSpec((2, BLK, GW), lambda i: (0, i, 0)),
            pl.BlockSpec((d_in, d_out), lambda i: (0, 0)),
            pl.BlockSpec((8, d_in), lambda i: (0, 0)),
        ],
        out_specs=[pl.BlockSpec((BLK, 32), lambda i: (i, 0))] * nq_out,
        out_shape=[jax.ShapeDtypeStruct((N_PAD, 32), jnp.float32)] * nq_out,
    )(p, *gs, dp, w, b8)


def _tc_last(p, g4, dp, b8, n_real):
    d_in = g4.shape[-1]

    def body(p_ref, g_ref, dp_ref, b_ref, h_ref, k_ref):
        i = pl.program_id(0)
        nd, _ = _norms_from_dp(dp_ref)
        agg = p_ref[0] + p_ref[1] + g_ref[...]
        h = jnp.maximum(agg * nd + b_ref[0:1, :], 0.0)
        h_ref[...] = h
        key = jnp.max(h, axis=1, keepdims=True)
        row = i * BLK + lax.broadcasted_iota(jnp.int32, (BLK, 1), 0)
        key = jnp.where(row < n_real, key, -1.0)
        k_ref[...] = jnp.broadcast_to(key, (BLK, 8))

    return pl.pallas_call(
        body,
        grid=(N_PAD // BLK,),
        in_specs=[
            pl.BlockSpec((2, BLK, d_in), lambda i: (0, i, 0)),
            pl.BlockSpec((BLK, d_in), lambda i: (i, 0)),
            pl.BlockSpec((2, BLK, GW), lambda i: (0, i, 0)),
            pl.BlockSpec((8, d_in), lambda i: (0, 0)),
        ],
        out_specs=[
            pl.BlockSpec((BLK, d_in), lambda i: (i, 0)),
            pl.BlockSpec((BLK, 8), lambda i: (i, 0)),
        ],
        out_shape=[
            jax.ShapeDtypeStruct((N_PAD, d_in), jnp.float32),
            jax.ShapeDtypeStruct((N_PAD, 8), jnp.float32),
        ],
    )(p, g4, dp, b8)


def _tc_pool(h4, key2d, k, d):
    rows2d, lanes = key2d.shape

    def body(h_ref, key_ref, o_ref, kbuf, gath):
        kbuf[...] = key_ref[...]
        flat = (lax.broadcasted_iota(jnp.int32, (rows2d, lanes), 0) * lanes
                + lax.broadcasted_iota(jnp.int32, (rows2d, lanes), 1))
        for j in range(k):
            kv = kbuf[...]
            m = jnp.max(kv)
            elig = kv == m
            node = jnp.min(jnp.where(elig, flat, jnp.int32(1 << 30)))
            kbuf[...] = jnp.where(flat == node, -2.0, kv)
            gath[pl.ds(j, 1), :] = h_ref[pl.ds(node, 1), :]
        rem = gath[...]
        lane = lax.broadcasted_iota(jnp.int32, (k, d), 1)
        big = jnp.float32(3.4e38)
        for j in range(d):
            cur = jnp.min(rem, axis=1, keepdims=True)
            o_ref[:, pl.ds(j, 1)] = cur
            pos = jnp.min(jnp.where(rem == cur, lane, jnp.int32(1 << 30)),
                          axis=1, keepdims=True)
            rem = jnp.where(lane == pos, big, rem)

    return pl.pallas_call(
        body,
        scratch_shapes=[
            pltpu.VMEM((rows2d, lanes), jnp.float32),
            pltpu.VMEM((k, d), jnp.float32),
        ],
        out_shape=jax.ShapeDtypeStruct((k, d), jnp.float32),
    )(h4, key2d)


def kernel(x, edge_index, W1, b1, W2, b2, W3, b3, W4, b4):
    n, in_dim = x.shape
    e = edge_index.shape[1]
    k = 16

    e_chunk = 1000
    e_pad = -(-e // (NS * e_chunk)) * (NS * e_chunk)
    src = edge_index[0]
    dst = edge_index[1]
    if e_pad != e:
        fill = jnp.full((e_pad - e,), n, jnp.int32)
        src = jnp.concatenate([src, fill])
        dst = jnp.concatenate([dst, fill])
    n_chunks = e_pad // (NS * e_chunk)

    xp = jnp.concatenate([x, jnp.zeros((N_PAD - n, in_dim), jnp.float32)])

    ones_tab = jnp.ones((N_PAD, GW), jnp.float32)
    dp = _aggregate((ones_tab, ones_tab), src, dst, e_chunk, n_chunks,
                    mode="role")

    ws = [W1, W2, W3, W4]
    bs = [jnp.broadcast_to(b[None, :], (8, b.shape[0])) for b in
          (b1, b2, b3, b4)]

    gs = _tc_first(xp, dp, ws[0])
    for layer in range(2):
        p = _aggregate(gs, src, dst, e_chunk, n_chunks)
        gs = _tc_mid(p, gs, dp, ws[layer + 1], bs[layer])
    p = _aggregate(gs, src, dst, e_chunk, n_chunks)
    (g4,) = _tc_mid(p, gs, dp, ws[3], bs[2])
    p = _aggregate((g4,), src, dst, e_chunk, n_chunks // NC, mode="edge")
    h4, key = _tc_last(p, g4, dp, bs[3], n)

    key2d = key[:, 0].reshape(N_PAD // 128, 128)
    pooled = _tc_pool(h4, key2d, k, h4.shape[-1])
    return pooled.reshape(1, k * h4.shape[-1])


# speedup vs baseline: 1.0204x; 1.0204x over previous
import functools
import jax
import jax.numpy as jnp
from jax import lax
from jax.experimental import pallas as pl
from jax.experimental.pallas import tpu as pltpu
from jax.experimental.pallas import tpu_sc as plsc

NC = 2
NS = 16
NW = NC * NS

N_PAD = 10240
ZR = N_PAD // NS
BLK = 2048
GW = 16


def _sc_mesh():
    return plsc.VectorSubcoreMesh(
        core_axis_name="c", subcore_axis_name="s", num_cores=NC, num_subcores=NS
    )


def _aggregate(gs, src, dst, e_chunk, n_chunks, mode="feature"):
    nq = len(gs)
    w = gs[0].shape[-1]
    per_core = nq // NC if mode == "feature" else 1
    n_out = nq if mode == "feature" else NC
    e_src = src.shape[0]
    e_half = e_src // NC
    zeros = jnp.zeros((ZR, w), jnp.float32)

    @functools.partial(
        pl.kernel,
        out_type=jax.ShapeDtypeStruct((n_out, N_PAD, w), jnp.float32),
        mesh=_sc_mesh(),
        scratch_types=[
            pltpu.VMEM((2, e_chunk), jnp.int32),
            pltpu.VMEM((2, e_chunk), jnp.int32),
            pltpu.VMEM((2, e_chunk, w), jnp.float32),
            pltpu.VMEM_SHARED((N_PAD, w), jnp.float32),
            pltpu.SemaphoreType.DMA,
            pltpu.SemaphoreType.DMA,
        ],
        compiler_params=pltpu.CompilerParams(use_tc_tiling_on_sc=False),
    )
    def agg_kernel(*refs):
        g_hbms = refs[:nq]
        (src_hbm, dst_hbm, z_hbm, out_hbm, idx_s, idx_d, rows_v, acc, sem0,
         sem1) = refs[nq:]
        sems = (sem0, sem1)
        c = lax.axis_index("c")
        s = lax.axis_index("s")
        base = s * (e_chunk * n_chunks)

        def offs(cc, ch):
            off = base + ch * e_chunk
            if mode == "edge":
                return cc * e_half + off, cc * e_half + off
            if mode == "role":
                return off, off
            return off, off

        for t in range(per_core):
            pltpu.sync_copy(z_hbm, acc.at[pl.ds(s * ZR, ZR)])
            plsc.subcore_barrier()
            for cc in range(NC):
                @pl.when(c == cc)
                def _():
                    s_off, d_off = offs(cc, 0)
                    pltpu.sync_copy(src_hbm.at[pl.ds(s_off, e_chunk)],
                                    idx_s.at[0])
                    d_ref0 = src_hbm if (mode == "role" and cc == 1) \
                        else dst_hbm
                    pltpu.sync_copy(d_ref0.at[pl.ds(d_off, e_chunk)],
                                    idx_d.at[0])
                    g = g_hbms[(cc * per_core + t) % nq]
                    pltpu.async_copy(g.at[idx_s.at[0]], rows_v.at[0],
                                     sems[0])
            for ch in range(n_chunks):
                b = 0 if mode == "role" else ch % 2
                nb = 1 - (ch % 2)
                for cc in range(NC):
                    @pl.when(c == cc)
                    def _():
                        g = g_hbms[(cc * per_core + t) % nq]
                        if mode == "role":
                            if ch == 0:
                                pltpu.make_async_copy(
                                    g.at[idx_s.at[0]], rows_v.at[0],
                                    sems[0]).wait()
                            if ch + 1 < n_chunks:
                                d_ref = dst_hbm if cc == 0 else src_hbm
                                off1 = base + (ch + 1) * e_chunk
                                pltpu.sync_copy(
                                    d_ref.at[pl.ds(off1, e_chunk)],
                                    idx_d.at[nb])
                        else:
                            if ch + 1 < n_chunks:
                                s_off, d_off = offs(cc, ch + 1)
                                pltpu.sync_copy(
                                    src_hbm.at[pl.ds(s_off, e_chunk)],
                                    idx_s.at[nb])
                                pltpu.sync_copy(
                                    dst_hbm.at[pl.ds(d_off, e_chunk)],
                                    idx_d.at[nb])
                                pltpu.async_copy(g.at[idx_s.at[nb]],
                                                 rows_v.at[nb], sems[nb])
                            pltpu.make_async_copy(g.at[idx_s.at[b]],
                                                  rows_v.at[b],
                                                  sems[b]).wait()

                db = b if mode != "role" else ch % 2
                pltpu.sync_copy(rows_v.at[b], acc.at[idx_d.at[db]],
                                add=True)
            plsc.subcore_barrier()
            for cc in range(NC):
                @pl.when(c == cc)
                def _():
                    oq = cc * per_core + t if mode == "feature" else cc
                    pltpu.sync_copy(acc.at[pl.ds(s * ZR, ZR)],
                                    out_hbm.at[oq, pl.ds(s * ZR, ZR)])

            if t + 1 < per_core:
                plsc.subcore_barrier()

    return agg_kernel(*gs, src, dst, zeros)


def _norms_from_dp(dp_ref):
    nd = lax.rsqrt(dp_ref[0, :, 0:1] + 1.0)
    ns = lax.rsqrt(dp_ref[1, :, 0:1] + 1.0)
    return nd, ns


def _grouped(o_refs, g):
    w = g.shape[-1] // len(o_refs)
    for q, o in enumerate(o_refs):
        o[...] = g[:, q * w:(q + 1) * w]


def _tc_first(xp, dp, w1, nq=None):
    in_dim, d_out = w1.shape
    if nq is None:
        nq = d_out // 32

    def body(x_ref, dp_ref, w_ref, *o_refs):
        _, ns = _norms_from_dp(dp_ref)
        g = jnp.dot(x_ref[...] * ns, w_ref[...],
                    preferred_element_type=jnp.float32)
        _grouped(o_refs, g)

    return pl.pallas_call(
        body,
        grid=(N_PAD // BLK,),
        in_specs=[
            pl.BlockSpec((BLK, in_dim), lambda i: (i, 0)),
            pl.BlockSpec((2, BLK, GW), lambda i: (0, i, 0)),
            pl.BlockSpec((in_dim, d_out), lambda i: (0, 0)),
        ],
        out_specs=[pl.BlockSpec((BLK, d_out // nq), lambda i: (i, 0))] * nq,
        out_shape=[jax.ShapeDtypeStruct((N_PAD, d_out // nq),
                                        jnp.float32)] * nq,
    )(xp, dp, w1)


def _tc_mid(p, gs, dp, w, b8):
    d_in, d_out = w.shape
    nq_in = len(gs)
    wi = d_in // nq_in
    nq_out = d_out // 32

    def body(p_ref, *refs):
        g_refs = refs[:nq_in]
        dp_ref, w_ref, b_ref = refs[nq_in:nq_in + 3]
        o_refs = refs[nq_in + 3:]
        nd, ns = _norms_from_dp(dp_ref)
        agg = jnp.concatenate(
            [p_ref[q] + g_refs[q][...] for q in range(nq_in)], axis=1)
        h = jnp.maximum(agg * nd + b_ref[0:1, :], 0.0)
        g = jnp.dot(h * ns, w_ref[...], preferred_element_type=jnp.float32)
        _grouped(o_refs, g)

    return pl.pallas_call(
        body,
        grid=(N_PAD // BLK,),
        in_specs=[pl.BlockSpec((nq_in, BLK, wi), lambda i: (0, i, 0))]
        + [pl.BlockSpec((BLK, wi), lambda i: (i, 0))] * nq_in
        + [
            pl.BlockSpec((2, BLK, GW), lambda i: (0, i, 0)),
            pl.BlockSpec((d_in, d_out), lambda i: (0, 0)),
            pl.BlockSpec((8, d_in), lambda i: (0, 0)),
        ],
        out_specs=[pl.BlockSpec((BLK, 32), lambda i: (i, 0))] * nq_out,
        out_shape=[jax.ShapeDtypeStruct((N_PAD, 32), jnp.float32)] * nq_out,
    )(p, *gs, dp, w, b8)


def _tc_mid2(p, gs, dp, w, b8):
    d_in, d_out = w.shape
    nq_out = d_out // 32

    def body(p_ref, g_ref, dp_ref, w_ref, b_ref, *o_refs):
        nd, ns = _norms_from_dp(dp_ref)
        agg = p_ref[0] + p_ref[1] + g_ref[...]
        h = jnp.maximum(agg * nd + b_ref[0:1, :], 0.0)
        g = jnp.dot(h * ns, w_ref[...], preferred_element_type=jnp.float32)
        _grouped(o_refs, g)

    return pl.pallas_call(
        body,
        grid=(N_PAD // BLK,),
        in_specs=[
            pl.BlockSpec((2, BLK, d_in), lambda i: (0, i, 0)),
            pl.BlockSpec((BLK, d_in), lambda i: (i, 0)),
            pl.BlockSpec((2, BLK, GW), lambda i: (0, i, 0)),
            pl.BlockSpec((d_in, d_out), lambda i: (0, 0)),
            pl.BlockSpec((8, d_in), lambda i: (0, 0)),
        ],
        out_specs=[pl.BlockSpec((BLK, 32), lambda i: (i, 0))] * nq_out,
        out_shape=[jax.ShapeDtypeStruct((N_PAD, 32), jnp.float32)] * nq_out,
    )(p, gs[0], dp, w, b8)


def _tc_last(p, g4, dp, b8, n_real):
    d_in = g4.shape[-1]

    def body(p_ref, g_ref, dp_ref, b_ref, h_ref, k_ref):
        i = pl.program_id(0)
        nd, _ = _norms_from_dp(dp_ref)
        agg = p_ref[0] + p_ref[1] + g_ref[...]
        h = jnp.maximum(agg * nd + b_ref[0:1, :], 0.0)
        h_ref[...] = h
        key = jnp.max(h, axis=1, keepdims=True)
        row = i * BLK + lax.broadcasted_iota(jnp.int32, (BLK, 1), 0)
        key = jnp.where(row < n_real, key, -1.0)
        k_ref[...] = jnp.broadcast_to(key, (BLK, 8))

    return pl.pallas_call(
        body,
        grid=(N_PAD // BLK,),
        in_specs=[
            pl.BlockSpec((2, BLK, d_in), lambda i: (0, i, 0)),
            pl.BlockSpec((BLK, d_in), lambda i: (i, 0)),
            pl.BlockSpec((2, BLK, GW), lambda i: (0, i, 0)),
            pl.BlockSpec((8, d_in), lambda i: (0, 0)),
        ],
        out_specs=[
            pl.BlockSpec((BLK, d_in), lambda i: (i, 0)),
            pl.BlockSpec((BLK, 8), lambda i: (i, 0)),
        ],
        out_shape=[
            jax.ShapeDtypeStruct((N_PAD, d_in), jnp.float32),
            jax.ShapeDtypeStruct((N_PAD, 8), jnp.float32),
        ],
    )(p, g4, dp, b8)


def _tc_pool(h4, key2d, k, d):
    rows2d, lanes = key2d.shape

    def body(h_ref, key_ref, o_ref, kbuf, gath):
        kbuf[...] = key_ref[...]
        flat = (lax.broadcasted_iota(jnp.int32, (rows2d, lanes), 0) * lanes
                + lax.broadcasted_iota(jnp.int32, (rows2d, lanes), 1))
        for j in range(k):
            kv = kbuf[...]
            m = jnp.max(kv)
            elig = kv == m
            node = jnp.min(jnp.where(elig, flat, jnp.int32(1 << 30)))
            kbuf[...] = jnp.where(flat == node, -2.0, kv)
            gath[pl.ds(j, 1), :] = h_ref[pl.ds(node, 1), :]
        rem = gath[...]
        lane = lax.broadcasted_iota(jnp.int32, (k, d), 1)
        big = jnp.float32(3.4e38)
        for j in range(d):
            cur = jnp.min(rem, axis=1, keepdims=True)
            o_ref[:, pl.ds(j, 1)] = cur
            pos = jnp.min(jnp.where(rem == cur, lane, jnp.int32(1 << 30)),
                          axis=1, keepdims=True)
            rem = jnp.where(lane == pos, big, rem)

    return pl.pallas_call(
        body,
        scratch_shapes=[
            pltpu.VMEM((rows2d, lanes), jnp.float32),
            pltpu.VMEM((k, d), jnp.float32),
        ],
        out_shape=jax.ShapeDtypeStruct((k, d), jnp.float32),
    )(h4, key2d)


def kernel(x, edge_index, W1, b1, W2, b2, W3, b3, W4, b4):
    n, in_dim = x.shape
    e = edge_index.shape[1]
    k = 16

    e_chunk = 1000
    e_pad = -(-e // (NS * e_chunk)) * (NS * e_chunk)
    src = edge_index[0]
    dst = edge_index[1]
    if e_pad != e:
        fill = jnp.full((e_pad - e,), n, jnp.int32)
        src = jnp.concatenate([src, fill])
        dst = jnp.concatenate([dst, fill])
    n_chunks = e_pad // (NS * e_chunk)

    xp = jnp.concatenate([x, jnp.zeros((N_PAD - n, in_dim), jnp.float32)])

    ones_tab = jnp.ones((N_PAD, GW), jnp.float32)
    dp = _aggregate((ones_tab, ones_tab), src, dst, e_chunk, n_chunks,
                    mode="role")

    ws = [W1, W2, W3, W4]
    bs = [jnp.broadcast_to(b[None, :], (8, b.shape[0])) for b in
          (b1, b2, b3, b4)]

    (g1,) = _tc_first(xp, dp, ws[0], nq=1)
    ec1 = 400
    p = _aggregate((g1,), src, dst, ec1, e_pad // (NC * NS * ec1),
                   mode="edge")
    gs = _tc_mid2(p, (g1,), dp, ws[1], bs[0])
    for layer in range(1, 2):
        p = _aggregate(gs, src, dst, e_chunk, n_chunks)
        gs = _tc_mid(p, gs, dp, ws[layer + 1], bs[layer])
    p = _aggregate(gs, src, dst, e_chunk, n_chunks)
    (g4,) = _tc_mid(p, gs, dp, ws[3], bs[2])
    p = _aggregate((g4,), src, dst, e_chunk, n_chunks // NC, mode="edge")
    h4, key = _tc_last(p, g4, dp, bs[3], n)

    key2d = key[:, 0].reshape(N_PAD // 128, 128)
    pooled = _tc_pool(h4, key2d, k, h4.shape[-1])
    return pooled.reshape(1, k * h4.shape[-1])


# speedup vs baseline: 1.0225x; 1.0021x over previous
import functools
import jax
import jax.numpy as jnp
from jax import lax
from jax.experimental import pallas as pl
from jax.experimental.pallas import tpu as pltpu
from jax.experimental.pallas import tpu_sc as plsc

NC = 2
NS = 16
NW = NC * NS

N_PAD = 10240
ZR = N_PAD // NS
BLK = 2048
GW = 16


def _sc_mesh():
    return plsc.VectorSubcoreMesh(
        core_axis_name="c", subcore_axis_name="s", num_cores=NC, num_subcores=NS
    )


def _aggregate(gs, src, dst, e_chunk, n_chunks, mode="feature"):
    nq = len(gs)
    w = gs[0].shape[-1]
    per_core = nq // NC if mode == "feature" else 1
    n_out = nq if mode == "feature" else NC
    e_src = src.shape[0]
    e_half = e_src // NC
    zeros = jnp.zeros((ZR, w), jnp.float32)

    @functools.partial(
        pl.kernel,
        out_type=jax.ShapeDtypeStruct((n_out, N_PAD, w), jnp.float32),
        mesh=_sc_mesh(),
        scratch_types=[
            pltpu.VMEM((2, e_chunk), jnp.int32),
            pltpu.VMEM((2, e_chunk), jnp.int32),
            pltpu.VMEM((2, e_chunk, w), jnp.float32),
            pltpu.VMEM_SHARED((N_PAD, w), jnp.float32),
            pltpu.SemaphoreType.DMA,
            pltpu.SemaphoreType.DMA,
        ],
        compiler_params=pltpu.CompilerParams(use_tc_tiling_on_sc=False),
    )
    def agg_kernel(*refs):
        g_hbms = refs[:nq]
        (src_hbm, dst_hbm, z_hbm, out_hbm, idx_s, idx_d, rows_v, acc, sem0,
         sem1) = refs[nq:]
        sems = (sem0, sem1)
        c = lax.axis_index("c")
        s = lax.axis_index("s")
        base = s * (e_chunk * n_chunks)

        def offs(cc, ch):
            off = base + ch * e_chunk
            if mode == "edge":
                return cc * e_half + off, cc * e_half + off
            if mode == "role":
                return off, off
            return off, off

        for t in range(per_core):
            pltpu.sync_copy(z_hbm, acc.at[pl.ds(s * ZR, ZR)])
            plsc.subcore_barrier()
            for cc in range(NC):
                @pl.when(c == cc)
                def _():
                    s_off, d_off = offs(cc, 0)
                    pltpu.sync_copy(src_hbm.at[pl.ds(s_off, e_chunk)],
                                    idx_s.at[0])
                    d_ref0 = src_hbm if (mode == "role" and cc == 1) \
                        else dst_hbm
                    pltpu.sync_copy(d_ref0.at[pl.ds(d_off, e_chunk)],
                                    idx_d.at[0])
                    g = g_hbms[(cc * per_core + t) % nq]
                    pltpu.async_copy(g.at[idx_s.at[0]], rows_v.at[0],
                                     sems[0])
            for ch in range(n_chunks):
                b = 0 if mode == "role" else ch % 2
                nb = 1 - (ch % 2)
                for cc in range(NC):
                    @pl.when(c == cc)
                    def _():
                        g = g_hbms[(cc * per_core + t) % nq]
                        if mode == "role":
                            if ch == 0:
                                pltpu.make_async_copy(
                                    g.at[idx_s.at[0]], rows_v.at[0],
                                    sems[0]).wait()
                            if ch + 1 < n_chunks:
                                d_ref = dst_hbm if cc == 0 else src_hbm
                                off1 = base + (ch + 1) * e_chunk
                                pltpu.sync_copy(
                                    d_ref.at[pl.ds(off1, e_chunk)],
                                    idx_d.at[nb])
                        else:
                            if ch + 1 < n_chunks:
                                s_off, d_off = offs(cc, ch + 1)
                                pltpu.sync_copy(
                                    src_hbm.at[pl.ds(s_off, e_chunk)],
                                    idx_s.at[nb])
                                pltpu.sync_copy(
                                    dst_hbm.at[pl.ds(d_off, e_chunk)],
                                    idx_d.at[nb])
                                pltpu.async_copy(g.at[idx_s.at[nb]],
                                                 rows_v.at[nb], sems[nb])
                            pltpu.make_async_copy(g.at[idx_s.at[b]],
                                                  rows_v.at[b],
                                                  sems[b]).wait()

                db = b if mode != "role" else ch % 2
                pltpu.sync_copy(rows_v.at[b], acc.at[idx_d.at[db]],
                                add=True)
            plsc.subcore_barrier()
            for cc in range(NC):
                @pl.when(c == cc)
                def _():
                    oq = cc * per_core + t if mode == "feature" else cc
                    pltpu.sync_copy(acc.at[pl.ds(s * ZR, ZR)],
                                    out_hbm.at[oq, pl.ds(s * ZR, ZR)])

            if t + 1 < per_core:
                plsc.subcore_barrier()

    return agg_kernel(*gs, src, dst, zeros)


def _norms_from_dp(dp_ref):
    nd = lax.rsqrt(dp_ref[0, :, 0:1] + 1.0)
    ns = lax.rsqrt(dp_ref[1, :, 0:1] + 1.0)
    return nd, ns


def _grouped(o_refs, g):
    w = g.shape[-1] // len(o_refs)
    for q, o in enumerate(o_refs):
        o[...] = g[:, q * w:(q + 1) * w]


def _tc_first(xp, dp, w1, xblk=BLK):
    in_dim, d_out = w1.shape
    nq = d_out // 32
    rows = xp.shape[0]

    def body(x_ref, dp_ref, w_ref, *o_refs):
        _, ns = _norms_from_dp(dp_ref)
        g = jnp.dot(x_ref[...] * ns, w_ref[...],
                    preferred_element_type=jnp.float32)
        _grouped(o_refs, g)

    return pl.pallas_call(
        body,
        grid=(rows // xblk,),
        in_specs=[
            pl.BlockSpec((xblk, in_dim), lambda i: (i, 0)),
            pl.BlockSpec((2, xblk, GW), lambda i: (0, i, 0)),
            pl.BlockSpec((in_dim, d_out), lambda i: (0, 0)),
        ],
        out_specs=[pl.BlockSpec((xblk, d_out // nq), lambda i: (i, 0))] * nq,
        out_shape=[jax.ShapeDtypeStruct((N_PAD, d_out // nq),
                                        jnp.float32)] * nq,
    )(xp, dp, w1)


def _tc_mid(p, gs, dp, w, b8):
    d_in, d_out = w.shape
    nq_in = len(gs)
    wi = d_in // nq_in
    nq_out = d_out // 32

    def body(p_ref, *refs):
        g_refs = refs[:nq_in]
        dp_ref, w_ref, b_ref = refs[nq_in:nq_in + 3]
        o_refs = refs[nq_in + 3:]
        nd, ns = _norms_from_dp(dp_ref)
        agg = jnp.concatenate(
            [p_ref[q] + g_refs[q][...] for q in range(nq_in)], axis=1)
        h = jnp.maximum(agg * nd + b_ref[0:1, :], 0.0)
        g = jnp.dot(h * ns, w_ref[...], preferred_element_type=jnp.float32)
        _grouped(o_refs, g)

    return pl.pallas_call(
        body,
        grid=(N_PAD // BLK,),
        in_specs=[pl.BlockSpec((nq_in, BLK, wi), lambda i: (0, i, 0))]
        + [pl.BlockSpec((BLK, wi), lambda i: (i, 0))] * nq_in
        + [
            pl.BlockSpec((2, BLK, GW), lambda i: (0, i, 0)),
            pl.BlockSpec((d_in, d_out), lambda i: (0, 0)),
            pl.BlockSpec((8, d_in), lambda i: (0, 0)),
        ],
        out_specs=[pl.BlockSpec((BLK, 32), lambda i: (i, 0))] * nq_out,
        out_shape=[jax.ShapeDtypeStruct((N_PAD, 32), jnp.float32)] * nq_out,
    )(p, *gs, dp, w, b8)


def _tc_mid2(p, gs, dp, w, b8):
    d_in, d_out = w.shape
    nq_out = d_out // 32

    def body(p_ref, g_ref, dp_ref, w_ref, b_ref, *o_refs):
        nd, ns = _norms_from_dp(dp_ref)
        agg = p_ref[0] + p_ref[1] + g_ref[...]
        h = jnp.maximum(agg * nd + b_ref[0:1, :], 0.0)
        g = jnp.dot(h * ns, w_ref[...], preferred_element_type=jnp.float32)
        _grouped(o_refs, g)

    return pl.pallas_call(
        body,
        grid=(N_PAD // BLK,),
        in_specs=[
            pl.BlockSpec((2, BLK, d_in), lambda i: (0, i, 0)),
            pl.BlockSpec((BLK, d_in), lambda i: (i, 0)),
            pl.BlockSpec((2, BLK, GW), lambda i: (0, i, 0)),
            pl.BlockSpec((d_in, d_out), lambda i: (0, 0)),
            pl.BlockSpec((8, d_in), lambda i: (0, 0)),
        ],
        out_specs=[pl.BlockSpec((BLK, 32), lambda i: (i, 0))] * nq_out,
        out_shape=[jax.ShapeDtypeStruct((N_PAD, 32), jnp.float32)] * nq_out,
    )(p, gs[0], dp, w, b8)


def _tc_last(p, g4, dp, b8, n_real):
    d_in = g4.shape[-1]

    def body(p_ref, g_ref, dp_ref, b_ref, h_ref, k_ref):
        i = pl.program_id(0)
        nd, _ = _norms_from_dp(dp_ref)
        agg = p_ref[0] + p_ref[1] + g_ref[...]
        h = jnp.maximum(agg * nd + b_ref[0:1, :], 0.0)
        h_ref[...] = h
        key = jnp.max(h, axis=1, keepdims=True)
        row = i * BLK + lax.broadcasted_iota(jnp.int32, (BLK, 1), 0)
        key = jnp.where(row < n_real, key, -1.0)
        k_ref[...] = jnp.broadcast_to(key, (BLK, 8))

    return pl.pallas_call(
        body,
        grid=(N_PAD // BLK,),
        in_specs=[
            pl.BlockSpec((2, BLK, d_in), lambda i: (0, i, 0)),
            pl.BlockSpec((BLK, d_in), lambda i: (i, 0)),
            pl.BlockSpec((2, BLK, GW), lambda i: (0, i, 0)),
            pl.BlockSpec((8, d_in), lambda i: (0, 0)),
        ],
        out_specs=[
            pl.BlockSpec((BLK, d_in), lambda i: (i, 0)),
            pl.BlockSpec((BLK, 8), lambda i: (i, 0)),
        ],
        out_shape=[
            jax.ShapeDtypeStruct((N_PAD, d_in), jnp.float32),
            jax.ShapeDtypeStruct((N_PAD, 8), jnp.float32),
        ],
    )(p, g4, dp, b8)


def _tc_pool(h4, key2d, k, d):
    rows2d, lanes = key2d.shape

    def body(h_ref, key_ref, o_ref, kbuf, gath):
        kbuf[...] = key_ref[...]
        flat = (lax.broadcasted_iota(jnp.int32, (rows2d, lanes), 0) * lanes
                + lax.broadcasted_iota(jnp.int32, (rows2d, lanes), 1))
        for j in range(k):
            kv = kbuf[...]
            m = jnp.max(kv)
            elig = kv == m
            node = jnp.min(jnp.where(elig, flat, jnp.int32(1 << 30)))
            kbuf[...] = jnp.where(flat == node, -2.0, kv)
            gath[pl.ds(j, 1), :] = h_ref[pl.ds(node, 1), :]
        rem = gath[...]
        lane = lax.broadcasted_iota(jnp.int32, (k, d), 1)
        big = jnp.float32(3.4e38)
        for j in range(d):
            cur = jnp.min(rem, axis=1, keepdims=True)
            o_ref[:, pl.ds(j, 1)] = cur
            pos = jnp.min(jnp.where(rem == cur, lane, jnp.int32(1 << 30)),
                          axis=1, keepdims=True)
            rem = jnp.where(lane == pos, big, rem)

    return pl.pallas_call(
        body,
        scratch_shapes=[
            pltpu.VMEM((rows2d, lanes), jnp.float32),
            pltpu.VMEM((k, d), jnp.float32),
        ],
        out_shape=jax.ShapeDtypeStruct((k, d), jnp.float32),
    )(h4, key2d)


def kernel(x, edge_index, W1, b1, W2, b2, W3, b3, W4, b4):
    n, in_dim = x.shape
    e = edge_index.shape[1]
    k = 16

    e_chunk = 1000
    e_pad = -(-e // (NS * e_chunk)) * (NS * e_chunk)
    src = edge_index[0]
    dst = edge_index[1]
    if e_pad != e:
        fill = jnp.full((e_pad - e,), n, jnp.int32)
        src = jnp.concatenate([src, fill])
        dst = jnp.concatenate([dst, fill])
    n_chunks = e_pad // (NS * e_chunk)

    if n % 2000 == 0:
        xp = x
        xblk = 2000
    else:
        xp = jnp.concatenate([x, jnp.zeros((N_PAD - n, in_dim),
                                           jnp.float32)])
        xblk = BLK

    ones_tab = jnp.ones((N_PAD, GW), jnp.float32)
    dp = _aggregate((ones_tab, ones_tab), src, dst, e_chunk, n_chunks,
                    mode="role")

    ws = [W1, W2, W3, W4]
    bs = [jnp.broadcast_to(b[None, :], (8, b.shape[0])) for b in
          (b1, b2, b3, b4)]

    gs = _tc_first(xp, dp, ws[0], xblk)
    for layer in range(2):
        p = _aggregate(gs, src, dst, e_chunk, n_chunks)
        gs = _tc_mid(p, gs, dp, ws[layer + 1], bs[layer])
    p = _aggregate(gs, src, dst, e_chunk, n_chunks)
    (g4,) = _tc_mid(p, gs, dp, ws[3], bs[2])
    p = _aggregate((g4,), src, dst, e_chunk, n_chunks // NC, mode="edge")
    h4, key = _tc_last(p, g4, dp, bs[3], n)

    key2d = key[:, 0].reshape(N_PAD // 128, 128)
    pooled = _tc_pool(h4, key2d, k, h4.shape[-1])
    return pooled.reshape(1, k * h4.shape[-1])
